# speedup vs baseline: 71.2565x; 71.2565x over previous
import jax
import jax.numpy as jnp
from jax import lax
from jax.experimental import pallas as pl

IOU_THR = 0.5


def _make_nms_body(B: int, nb: int):
    def body(rows_ref, cols_ref, keep_ref):
        def iou_over(bj_start, x1r, y1r, x2r, y2r, arear):
            x1c = cols_ref[pl.ds(bj_start, B), 0:1]
            y1c = cols_ref[pl.ds(bj_start, B), 1:2]
            x2c = cols_ref[pl.ds(bj_start, B), 2:3]
            y2c = cols_ref[pl.ds(bj_start, B), 3:4]
            areac = (x2c - x1c) * (y2c - y1c)
            w = jnp.maximum(jnp.minimum(x2c, x2r) - jnp.maximum(x1c, x1r), 0.0)
            h = jnp.maximum(jnp.minimum(y2c, y2r) - jnp.maximum(y1c, y1r), 0.0)
            inter = w * h
            union = areac + arear - inter
            iou = inter / (union + 1e-8)
            return (iou > IOU_THR).astype(jnp.float32)

        def process_block(bi, _):
            base = bi * B
            x1r = rows_ref[0:1, pl.ds(base, B)]
            y1r = rows_ref[1:2, pl.ds(base, B)]
            x2r = rows_ref[2:3, pl.ds(base, B)]
            y2r = rows_ref[3:4, pl.ds(base, B)]
            arear = (x2r - x1r) * (y2r - y1r)

            def cross(bj, supp):
                over = iou_over(bj * B, x1r, y1r, x2r, y2r, arear)
                kprev = keep_ref[0:1, pl.ds(bj * B, B)]
                return supp + lax.dot_general(
                    kprev, over, (((1,), (0,)), ((), ())),
                    preferred_element_type=jnp.float32)

            supp = lax.fori_loop(0, bi, cross, jnp.zeros((1, B), jnp.float32))
            active = (supp < 0.5).astype(jnp.float32)

            over_d = iou_over(base, x1r, y1r, x2r, y2r, arear)
            ri = lax.broadcasted_iota(jnp.int32, (B, B), 0)
            ci = lax.broadcasted_iota(jnp.int32, (B, B), 1)
            tri = over_d * (ri < ci).astype(jnp.float32)

            def cond(c):
                it, _, changed = c
                return jnp.logical_and(changed, it < B)

            def sweep(c):
                it, keep, _ = c
                s = lax.dot_general(
                    keep, tri, (((1,), (0,)), ((), ())),
                    preferred_element_type=jnp.float32)
                new = jnp.where(s > 0.5, 0.0, active)
                return it + 1, new, jnp.any(new != keep)

            _, keep_blk, _ = lax.while_loop(
                cond, sweep, (jnp.int32(0), active, jnp.bool_(True)))
            keep_ref[0:1, pl.ds(base, B)] = keep_blk
            return 0

        lax.fori_loop(0, nb, process_block, 0)

    return body


@jax.jit
def kernel(boxes, scores):
    n = boxes.shape[0]
    B = 512
    nb = -(-n // B)
    npad = nb * B

    scores_p = jnp.concatenate(
        [scores, jnp.full((npad - n,), -1.0, scores.dtype)])
    boxes_p = jnp.concatenate(
        [boxes, jnp.zeros((npad - n, 4), boxes.dtype)])

    order = jnp.argsort(-scores_p)
    sboxes = boxes_p[order]

    rows = jnp.zeros((8, npad), jnp.float32).at[0:4, :].set(sboxes.T)

    keep_s = pl.pallas_call(
        _make_nms_body(B, nb),
        out_shape=jax.ShapeDtypeStruct((1, npad), jnp.float32),
    )(rows, sboxes)

    keep_sorted = keep_s[0] > 0.5
    keep = jnp.zeros((npad,), bool).at[order].set(keep_sorted)[:n]
    kept_scores = scores * keep.astype(scores.dtype)
    return keep, kept_scores


# speedup vs baseline: 95.2674x; 1.3370x over previous
import jax
import jax.numpy as jnp
from jax import lax
from jax.experimental import pallas as pl

IOU_THR = 0.5


def _make_nms_body(B: int, nb: int):
    def body(rows_ref, cols_ref, keep_ref):
        def process_block(bi, _):
            base = bi * B
            x1c = jnp.broadcast_to(cols_ref[pl.ds(base, B), 0:1], (B, B))
            y1c = jnp.broadcast_to(cols_ref[pl.ds(base, B), 1:2], (B, B))
            x2c = jnp.broadcast_to(cols_ref[pl.ds(base, B), 2:3], (B, B))
            y2c = jnp.broadcast_to(cols_ref[pl.ds(base, B), 3:4], (B, B))
            areac = (x2c - x1c) * (y2c - y1c)

            def over_tile(bj_start):
                x1r = rows_ref[0:1, pl.ds(bj_start, B)]
                y1r = rows_ref[1:2, pl.ds(bj_start, B)]
                x2r = rows_ref[2:3, pl.ds(bj_start, B)]
                y2r = rows_ref[3:4, pl.ds(bj_start, B)]
                arear = (x2r - x1r) * (y2r - y1r)
                w = jnp.maximum(
                    jnp.minimum(x2c, x2r) - jnp.maximum(x1c, x1r), 0.0)
                h = jnp.maximum(
                    jnp.minimum(y2c, y2r) - jnp.maximum(y1c, y1r), 0.0)
                inter = w * h
                union = areac + arear - inter
                iou = inter / (union + 1e-8)
                return (iou > IOU_THR).astype(jnp.float32)

            def cross(bj, supp):
                over = over_tile(bj * B)
                kprev = keep_ref[pl.ds(bj * B, B), 0:1]
                return supp + lax.dot_general(
                    over, kprev, (((1,), (0,)), ((), ())),
                    preferred_element_type=jnp.float32)

            supp = lax.fori_loop(0, bi, cross, jnp.zeros((B, 1), jnp.float32))
            active = (supp < 0.5).astype(jnp.float32)

            over_d = over_tile(base)
            ri = lax.broadcasted_iota(jnp.int32, (B, B), 0)
            ci = lax.broadcasted_iota(jnp.int32, (B, B), 1)
            tri = over_d * (ci < ri).astype(jnp.float32)

            def cond(c):
                it, _, changed = c
                return jnp.logical_and(changed, it < B)

            def sweep(c):
                it, keep, _ = c
                s = lax.dot_general(
                    tri, keep, (((1,), (0,)), ((), ())),
                    preferred_element_type=jnp.float32)
                new = jnp.where(s > 0.5, 0.0, active)
                return it + 1, new, jnp.any(new != keep)

            _, keep_blk, _ = lax.while_loop(
                cond, sweep, (jnp.int32(0), active, jnp.bool_(True)))
            keep_ref[pl.ds(base, B), 0:1] = keep_blk
            return 0

        lax.fori_loop(0, nb, process_block, 0)

    return body


@jax.jit
def kernel(boxes, scores):
    n = boxes.shape[0]
    B = 512
    nb = -(-n // B)
    npad = nb * B

    scores_p = jnp.concatenate(
        [scores, jnp.full((npad - n,), -1.0, scores.dtype)])
    boxes_p = jnp.concatenate(
        [boxes, jnp.zeros((npad - n, 4), boxes.dtype)])

    order = jnp.argsort(-scores_p)
    sboxes = boxes_p[order]

    rows = jnp.zeros((8, npad), jnp.float32).at[0:4, :].set(sboxes.T)

    keep_s = pl.pallas_call(
        _make_nms_body(B, nb),
        out_shape=jax.ShapeDtypeStruct((npad, 1), jnp.float32),
    )(rows, sboxes)

    keep_sorted = keep_s[:, 0] > 0.5
    keep = jnp.zeros((npad,), bool).at[order].set(keep_sorted)[:n]
    kept_scores = scores * keep.astype(scores.dtype)
    return keep, kept_scores


# speedup vs baseline: 101.6581x; 1.0671x over previous
import jax
import jax.numpy as jnp
from jax import lax
from jax.experimental import pallas as pl

IOU_THR = 0.5


def _make_nms_body(B: int, nb: int):
    def body(rows_ref, cols_ref, keep_ref):
        def process_block(bi, _):
            base = bi * B
            x1c = jnp.broadcast_to(cols_ref[pl.ds(base, B), 0:1], (B, B))
            y1c = jnp.broadcast_to(cols_ref[pl.ds(base, B), 1:2], (B, B))
            x2c = jnp.broadcast_to(cols_ref[pl.ds(base, B), 2:3], (B, B))
            y2c = jnp.broadcast_to(cols_ref[pl.ds(base, B), 3:4], (B, B))
            areac = (x2c - x1c) * (y2c - y1c)

            def over_tile(bj_start):
                x1r = rows_ref[0:1, pl.ds(bj_start, B)]
                y1r = rows_ref[1:2, pl.ds(bj_start, B)]
                x2r = rows_ref[2:3, pl.ds(bj_start, B)]
                y2r = rows_ref[3:4, pl.ds(bj_start, B)]
                arear = (x2r - x1r) * (y2r - y1r)
                w = jnp.maximum(
                    jnp.minimum(x2c, x2r) - jnp.maximum(x1c, x1r), 0.0)
                h = jnp.maximum(
                    jnp.minimum(y2c, y2r) - jnp.maximum(y1c, y1r), 0.0)
                inter = w * h
                union = areac + arear - inter
                iou = inter / (union + 1e-8)
                return (iou > IOU_THR).astype(jnp.float32)

            def cross(bj, supp):
                over = over_tile(bj * B)
                kprev = keep_ref[pl.ds(bj * B, B), 0:1]
                return supp + lax.dot_general(
                    over, kprev, (((1,), (0,)), ((), ())),
                    preferred_element_type=jnp.float32)

            supp = lax.fori_loop(0, bi, cross, jnp.zeros((B, 1), jnp.float32))
            active = (supp < 0.5).astype(jnp.float32)

            over_d = over_tile(base)
            ri = lax.broadcasted_iota(jnp.int32, (B, B), 0)
            ci = lax.broadcasted_iota(jnp.int32, (B, B), 1)
            tri = over_d * (ci < ri).astype(jnp.float32)

            def cond(c):
                it, _, changed = c
                return jnp.logical_and(changed, it < B)

            def sweep(c):
                it, keep, _ = c
                s = lax.dot_general(
                    tri, keep, (((1,), (0,)), ((), ())),
                    preferred_element_type=jnp.float32)
                new = jnp.where(s > 0.5, 0.0, active)
                return it + 1, new, jnp.any(new != keep)

            _, keep_blk, _ = lax.while_loop(
                cond, sweep, (jnp.int32(0), active, jnp.bool_(True)))
            keep_ref[pl.ds(base, B), 0:1] = keep_blk
            return 0

        lax.fori_loop(0, nb, process_block, 0)

    return body


@jax.jit
def kernel(boxes, scores):
    n = boxes.shape[0]
    B = 1024
    nb = -(-n // B)
    npad = nb * B

    scores_p = jnp.concatenate(
        [scores, jnp.full((npad - n,), -1.0, scores.dtype)])
    boxes_p = jnp.concatenate(
        [boxes, jnp.zeros((npad - n, 4), boxes.dtype)])

    order = jnp.argsort(-scores_p)
    sboxes = boxes_p[order]

    rows = jnp.zeros((8, npad), jnp.float32).at[0:4, :].set(sboxes.T)

    keep_s = pl.pallas_call(
        _make_nms_body(B, nb),
        out_shape=jax.ShapeDtypeStruct((npad, 1), jnp.float32),
    )(rows, sboxes)

    keep_sorted = keep_s[:, 0] > 0.5
    keep = jnp.zeros((npad,), bool).at[order].set(keep_sorted)[:n]
    kept_scores = scores * keep.astype(scores.dtype)
    return keep, kept_scores
